# baseline (speedup 1.0000x reference)
import functools

import jax
import jax.numpy as jnp
from jax import lax
from jax.experimental import pallas as pl
from jax.experimental.pallas import tpu as pltpu
from jax.experimental.pallas import tpu_sc as plsc

N_NODES = 10000
DIMS = 128
N_EDGES = 320000
R = 2.0
T = 1.0

NC = 2
NS = 16
NW = NC * NS
E_W = N_EDGES // NW
C = 40
N_CHUNKS = E_W // C
NBUF = 4
TAIL = N_CHUNKS % NBUF
L = 16
GROUPS = (C + L - 1) // L
assert E_W % C == 0 and C >= L and C % 8 == 0 and TAIL < NBUF
NSEG = DIMS // L

_mesh = plsc.VectorSubcoreMesh(core_axis_name="c", subcore_axis_name="s")


@functools.partial(
    pl.kernel,
    out_type=jax.ShapeDtypeStruct((N_EDGES,), jnp.float32),
    mesh=_mesh,
    scratch_types=[
        pltpu.VMEM((E_W,), jnp.int32),
        pltpu.VMEM((E_W,), jnp.int32),
        [pltpu.VMEM((C,), jnp.float32)] * NBUF,
        [pltpu.VMEM((C, DIMS), jnp.float32)] * NBUF,
        [pltpu.VMEM((C, DIMS), jnp.float32)] * NBUF,
        [pltpu.SemaphoreType.DMA] * NBUF,
        [pltpu.SemaphoreType.DMA] * NBUF,
        [pltpu.SemaphoreType.DMA] * NBUF,
    ],
    compiler_params=pltpu.CompilerParams(needs_layout_passes=False),
)
def _lp_decode(table, eidx, out, idx_u, idx_v, outb, rows_u, rows_v,
               sem_u, sem_v, sem_o):
    wid = lax.axis_index("s") * NC + lax.axis_index("c")
    base = pl.multiple_of(wid * E_W, 8)

    pltpu.sync_copy(eidx.at[pl.ds(base, E_W)], idx_u)
    pltpu.sync_copy(eidx.at[pl.ds(N_EDGES + base, E_W)], idx_v)

    def issue(ch, b):
        off = pl.multiple_of(ch * C, 8)
        pltpu.async_copy(table.at[idx_u.at[pl.ds(off, C)]], rows_u[b], sem_u[b])
        pltpu.async_copy(table.at[idx_v.at[pl.ds(off, C)]], rows_v[b], sem_v[b])

    def wait(ch, b):
        off = pl.multiple_of(ch * C, 8)
        pltpu.make_async_copy(table.at[idx_u.at[pl.ds(off, C)]], rows_u[b], sem_u[b]).wait()
        pltpu.make_async_copy(table.at[idx_v.at[pl.ds(off, C)]], rows_v[b], sem_v[b]).wait()

    row_ids = lax.iota(jnp.int32, L)

    def compute(ch, b):
        ru = rows_u[b]
        rv = rows_v[b]

        def group_body(g, _):
            e0 = jnp.minimum(g * L, C - L)
            tot = jnp.zeros((L,), jnp.float32)
            for i in range(L):
                e = e0 + i
                acc = None
                for k in range(NSEG):
                    du = ru[e, pl.ds(k * L, L)] - rv[e, pl.ds(k * L, L)]
                    sq = du * du
                    acc = sq if acc is None else acc + sq
                s = jnp.sum(acc)
                tot = jnp.where(row_ids == i, s, tot)
            probs = 1.0 / (jnp.exp((tot - R) / T) + 1.0)
            outb[b][pl.ds(e0, L)] = probs
            return 0

        lax.fori_loop(0, GROUPS, group_body, 0)

    def write_out(ch, b):
        off = pl.multiple_of(base + ch * C, 8)
        pltpu.async_copy(outb[b], out.at[pl.ds(off, C)], sem_o[b])

    def wait_out(ch, b):
        off = pl.multiple_of(base + ch * C, 8)
        pltpu.make_async_copy(outb[b], out.at[pl.ds(off, C)], sem_o[b]).wait()

    def turn(ch, b, static_last=False):
        wait(ch, b)
        if not static_last:
            nxt = ch + NBUF - 1

            @pl.when(nxt < N_CHUNKS)
            def _():
                issue(nxt, (b + NBUF - 1) % NBUF)

        if static_last:
            wait_out(ch - NBUF, b)
        else:
            @pl.when(ch >= NBUF)
            def _():
                wait_out(ch - NBUF, b)

        compute(ch, b)
        write_out(ch, b)

    for j in range(NBUF - 1):
        issue(j, j)

    def ring_body(m, _):
        for b in range(NBUF):
            turn(m * NBUF + b, b)
        return 0

    lax.fori_loop(0, (N_CHUNKS - TAIL) // NBUF, ring_body, 0)
    for t in range(TAIL):
        turn(N_CHUNKS - TAIL + t, t, static_last=True)
    for j in range(NBUF):
        ch = N_CHUNKS - NBUF + j
        wait_out(ch, ch % NBUF)


def kernel(node_features, edge_label_index):
    return _lp_decode(node_features, edge_label_index.reshape(-1))
